# Initial kernel scaffold; baseline (speedup 1.0000x reference)
import jax
import jax.numpy as jnp
from jax.experimental import pallas as pl


def kernel(inputs, edge_index, W0, W1, W2, epoch):
    raise NotImplementedError("write your pallas kernel here")


# speedup vs baseline: 4.5914x; 4.5914x over previous
import functools

import jax
import jax.numpy as jnp
from jax import lax
from jax.experimental import pallas as pl
from jax.experimental.pallas import tpu as pltpu
from jax.experimental.pallas import tpu_sc as plsc

N = 10000
E = 160000
F_IN = 256
HID = 256
CLS = 64

K = 128
NCHUNK = E // K
NSUB = 16
ROWS_PER_SUB = 624
ZROWS = 104
MBLK = 1000


def _make_spmm(dh):
    mesh = plsc.VectorSubcoreMesh(core_axis_name="c", subcore_axis_name="s")
    base_cnt = NCHUNK // NSUB
    extra = NCHUNK % NSUB

    @functools.partial(
        pl.kernel,
        out_type=(jax.ShapeDtypeStruct((N, dh), jnp.float32),
                  jax.ShapeDtypeStruct((N, dh), jnp.float32)),
        mesh=mesh,
        scratch_types=[
            pltpu.VMEM((K,), jnp.int32),
            pltpu.VMEM((K,), jnp.int32),
            pltpu.VMEM((K, dh), jnp.float32),
            pltpu.VMEM((ZROWS, dh), jnp.float32),
            pltpu.VMEM_SHARED((N, dh), jnp.float32),
            pltpu.SemaphoreType.DMA,
        ],
    )
    def spmm(h0, h1, src, dst, out0, out1, idx_s, idx_d, rows, zbuf, acc, sem):
        c = lax.axis_index("c")
        s = lax.axis_index("s")

        def zrow(r, carry):
            for j in range(dh // 16):
                zbuf[r, pl.ds(j * 16, 16)] = jnp.zeros((16,), jnp.float32)
            return carry
        lax.fori_loop(0, ZROWS, zrow, 0)
        r0 = s * ROWS_PER_SUB
        for t in range(ROWS_PER_SUB // ZROWS):
            pltpu.sync_copy(zbuf, acc.at[pl.ds(r0 + t * ZROWS, ZROWS)])

        @pl.when(s == NSUB - 1)
        def _():
            pltpu.sync_copy(zbuf.at[pl.ds(0, 16)],
                            acc.at[pl.ds(NSUB * ROWS_PER_SUB, 16)])
        plsc.subcore_barrier()

        cnt = jnp.where(s < extra, base_cnt + 1, base_cnt)
        start = s * base_cnt + jnp.minimum(s, extra)

        def do_half(h_ref):
            def body(j, carry):
                base = pl.multiple_of((start + j) * K, K)
                pltpu.sync_copy(src.at[pl.ds(base, K)], idx_s)
                pltpu.sync_copy(dst.at[pl.ds(base, K)], idx_d)
                pltpu.async_copy(h_ref.at[idx_s], rows, sem).wait()
                pltpu.sync_copy(rows, acc.at[idx_d], add=True)
                return carry
            lax.fori_loop(0, cnt, body, 0)

        @pl.when(c == 0)
        def _():
            do_half(h0)

        @pl.when(c == 1)
        def _():
            do_half(h1)

        plsc.subcore_barrier()

        def writeback(out_ref):
            for t in range(ROWS_PER_SUB // ZROWS):
                sl = pl.ds(r0 + t * ZROWS, ZROWS)
                pltpu.sync_copy(acc.at[sl], out_ref.at[sl])

            @pl.when(s == NSUB - 1)
            def _():
                sl = pl.ds(NSUB * ROWS_PER_SUB, 16)
                pltpu.sync_copy(acc.at[sl], out_ref.at[sl])

        @pl.when(c == 0)
        def _():
            writeback(out0)

        @pl.when(c == 1)
        def _():
            writeback(out1)

    return spmm


_spmm128 = _make_spmm(128)


def _make_spmm_edge_split():
    mesh = plsc.VectorSubcoreMesh(core_axis_name="c", subcore_axis_name="s")
    half_chunks = NCHUNK // 2
    base_cnt = half_chunks // NSUB
    extra = half_chunks % NSUB
    dh = 128

    @functools.partial(
        pl.kernel,
        out_type=(jax.ShapeDtypeStruct((N, dh), jnp.float32),
                  jax.ShapeDtypeStruct((N, dh), jnp.float32)),
        mesh=mesh,
        scratch_types=[
            pltpu.VMEM((K,), jnp.int32),
            pltpu.VMEM((K,), jnp.int32),
            pltpu.VMEM((K, dh), jnp.float32),
            pltpu.VMEM((ZROWS, dh), jnp.float32),
            pltpu.VMEM_SHARED((N, dh), jnp.float32),
            pltpu.SemaphoreType.DMA,
        ],
    )
    def spmm(h, src, dst, out0, out1, idx_s, idx_d, rows, zbuf, acc, sem):
        c = lax.axis_index("c")
        s = lax.axis_index("s")

        def zrow(r, carry):
            for j in range(dh // 16):
                zbuf[r, pl.ds(j * 16, 16)] = jnp.zeros((16,), jnp.float32)
            return carry
        lax.fori_loop(0, ZROWS, zrow, 0)
        r0 = s * ROWS_PER_SUB
        for t in range(ROWS_PER_SUB // ZROWS):
            pltpu.sync_copy(zbuf, acc.at[pl.ds(r0 + t * ZROWS, ZROWS)])

        @pl.when(s == NSUB - 1)
        def _():
            pltpu.sync_copy(zbuf.at[pl.ds(0, 16)],
                            acc.at[pl.ds(NSUB * ROWS_PER_SUB, 16)])
        plsc.subcore_barrier()

        cnt = jnp.where(s < extra, base_cnt + 1, base_cnt)
        start = c * half_chunks + s * base_cnt + jnp.minimum(s, extra)

        def body(j, carry):
            base = pl.multiple_of((start + j) * K, K)
            pltpu.sync_copy(src.at[pl.ds(base, K)], idx_s)
            pltpu.sync_copy(dst.at[pl.ds(base, K)], idx_d)
            pltpu.async_copy(h.at[idx_s], rows, sem).wait()
            pltpu.sync_copy(rows, acc.at[idx_d], add=True)
            return carry
        lax.fori_loop(0, cnt, body, 0)

        plsc.subcore_barrier()

        def writeback(out_ref):
            for t in range(ROWS_PER_SUB // ZROWS):
                sl = pl.ds(r0 + t * ZROWS, ZROWS)
                pltpu.sync_copy(acc.at[sl], out_ref.at[sl])

            @pl.when(s == NSUB - 1)
            def _():
                sl = pl.ds(NSUB * ROWS_PER_SUB, 16)
                pltpu.sync_copy(acc.at[sl], out_ref.at[sl])

        @pl.when(c == 0)
        def _():
            writeback(out0)

        @pl.when(c == 1)
        def _():
            writeback(out1)

    return spmm


_spmm_last = _make_spmm_edge_split()


def _gemm0(x, w):
    def body(x_ref, w_ref, oa, ob):
        h = jnp.dot(x_ref[...], w_ref[...], preferred_element_type=jnp.float32)
        oa[...] = h[:, :128]
        ob[...] = h[:, 128:]
    return pl.pallas_call(
        body,
        grid=(N // MBLK,),
        in_specs=[pl.BlockSpec((MBLK, F_IN), lambda i: (i, 0)),
                  pl.BlockSpec((F_IN, HID), lambda i: (0, 0))],
        out_specs=[pl.BlockSpec((MBLK, 128), lambda i: (i, 0))] * 2,
        out_shape=[jax.ShapeDtypeStruct((N, 128), jnp.float32)] * 2,
    )(x, w)


def _gemm_relu(ha, hb, w, dout):
    dh = dout // 2

    def body(a_ref, b_ref, w_ref, oa, ob):
        xa = jnp.maximum(a_ref[...], 0.0)
        xb = jnp.maximum(b_ref[...], 0.0)
        h = (jnp.dot(xa, w_ref[:128, :], preferred_element_type=jnp.float32)
             + jnp.dot(xb, w_ref[128:, :], preferred_element_type=jnp.float32))
        oa[...] = h[:, :dh]
        ob[...] = h[:, dh:]

    return pl.pallas_call(
        body,
        grid=(N // MBLK,),
        in_specs=[pl.BlockSpec((MBLK, 128), lambda i: (i, 0)),
                  pl.BlockSpec((MBLK, 128), lambda i: (i, 0)),
                  pl.BlockSpec((HID, dout), lambda i: (0, 0))],
        out_specs=[pl.BlockSpec((MBLK, dh), lambda i: (i, 0))] * 2,
        out_shape=[jax.ShapeDtypeStruct((N, dh), jnp.float32)] * 2,
    )(ha, hb, w)


def _gemm_relu_wide(ha, hb, w):
    def body(a_ref, b_ref, w_ref, o_ref):
        xa = jnp.maximum(a_ref[...], 0.0)
        xb = jnp.maximum(b_ref[...], 0.0)
        o_ref[...] = (
            jnp.dot(xa, w_ref[:128, :], preferred_element_type=jnp.float32)
            + jnp.dot(xb, w_ref[128:, :], preferred_element_type=jnp.float32))

    return pl.pallas_call(
        body,
        grid=(N // MBLK,),
        in_specs=[pl.BlockSpec((MBLK, 128), lambda i: (i, 0)),
                  pl.BlockSpec((MBLK, 128), lambda i: (i, 0)),
                  pl.BlockSpec((HID, 128), lambda i: (0, 0))],
        out_specs=pl.BlockSpec((MBLK, 128), lambda i: (i, 0)),
        out_shape=jax.ShapeDtypeStruct((N, 128), jnp.float32),
    )(ha, hb, w)


def _log_softmax_sum(p0, p1):
    def body(a_ref, b_ref, o_ref):
        x = (a_ref[...] + b_ref[...])[:, :CLS]
        m = jnp.max(x, axis=1, keepdims=True)
        sh = x - m
        o_ref[...] = sh - jnp.log(jnp.sum(jnp.exp(sh), axis=1, keepdims=True))

    return pl.pallas_call(
        body,
        grid=(N // MBLK,),
        in_specs=[pl.BlockSpec((MBLK, 128), lambda i: (i, 0)),
                  pl.BlockSpec((MBLK, 128), lambda i: (i, 0))],
        out_specs=pl.BlockSpec((MBLK, CLS), lambda i: (i, 0)),
        out_shape=jax.ShapeDtypeStruct((N, CLS), jnp.float32),
    )(p0, p1)


def kernel(inputs, edge_index, W0, W1, W2, epoch):
    src = edge_index[0]
    dst = edge_index[1]
    w2p = jnp.pad(W2, ((0, 0), (0, 128 - CLS)))
    h0a, h0b = _gemm0(inputs, W0)
    a0a, a0b = _spmm128(h0a, h0b, src, dst)
    h1a, h1b = _gemm_relu(a0a, a0b, W1, HID)
    a1a, a1b = _spmm128(h1a, h1b, src, dst)
    h2 = _gemm_relu_wide(a1a, a1b, w2p)
    p0, p1 = _spmm_last(h2, src, dst)
    return _log_softmax_sum(p0, p1)
